# baseline (speedup 1.0000x reference)
import functools

import jax
import jax.numpy as jnp
from jax import lax
from jax.experimental import pallas as pl
from jax.experimental.pallas import tpu as pltpu
from jax.experimental.pallas import tpu_sc as plsc

NC, NS, L = 2, 16, 16
NW = NC * NS

N_NODES = 10000
N_EDGES = 320000
D_FEAT = 128
D_OUT = 128

EPW = N_EDGES // NW
CH = 80
NCH = EPW // CH
RPS = 624
TAIL = N_NODES - NS * RPS
ZR = 78
NB = 10
BLK = N_NODES // NB

_mesh = plsc.VectorSubcoreMesh(core_axis_name="c", subcore_axis_name="s")
_sc_params = pltpu.CompilerParams(needs_layout_passes=False)


@functools.partial(
    pl.kernel,
    out_type=(
        jax.ShapeDtypeStruct((NB, NW, BLK), jnp.float32),
        jax.ShapeDtypeStruct((NB, NW, BLK), jnp.float32),
    ),
    mesh=_mesh,
    scratch_types=[
        pltpu.VMEM((EPW,), jnp.int32),
        pltpu.VMEM((EPW,), jnp.int32),
        pltpu.VMEM((EPW,), jnp.float32),
        pltpu.VMEM((N_NODES,), jnp.float32),
        pltpu.VMEM((N_NODES,), jnp.float32),
    ],
    compiler_params=pltpu.CompilerParams(
        needs_layout_passes=False, use_tc_tiling_on_sc=False),
)
def _deg_kernel(recv_hbm, send_hbm, w_hbm, histr_hbm, hists_hbm,
                ridx_v, sidx_v, w_v, hr, hs):
    c = lax.axis_index("c")
    s = lax.axis_index("s")
    wid = c * NS + s
    base = wid * EPW
    pltpu.sync_copy(recv_hbm.at[pl.ds(base, EPW)], ridx_v)
    pltpu.sync_copy(send_hbm.at[pl.ds(base, EPW)], sidx_v)
    pltpu.sync_copy(w_hbm.at[pl.ds(base, EPW)], w_v)

    zero = jnp.zeros((L,), jnp.float32)

    @pl.loop(0, N_NODES // L)
    def _zero(i):
        hr[pl.ds(i * L, L)] = zero
        hs[pl.ds(i * L, L)] = zero

    @pl.loop(0, EPW // L)
    def _acc(i):
        sl = pl.ds(i * L, L)
        wv = w_v[sl]
        plsc.addupdate_scatter(hr, [ridx_v[sl]], wv)
        plsc.addupdate_scatter(hs, [sidx_v[sl]], wv)

    for i in range(NB):
        pltpu.sync_copy(hr.at[pl.ds(i * BLK, BLK)], histr_hbm.at[i, wid])
        pltpu.sync_copy(hs.at[pl.ds(i * BLK, BLK)], hists_hbm.at[i, wid])


def _norm_body(hr_ref, hs_ref, x_ref, xs_ref, invr_ref):
    deg_r = jnp.sum(hr_ref[0], axis=0)
    safe_r = jnp.where(deg_r > 0, deg_r, 1.0)
    invr_ref[...] = jnp.where(deg_r > 0, lax.rsqrt(safe_r), 0.0)[None, None, :]
    deg_s = jnp.sum(hs_ref[0], axis=0)
    safe_s = jnp.where(deg_s > 0, deg_s, 1.0)
    inv_s = jnp.where(deg_s > 0, lax.rsqrt(safe_s), 0.0)
    xs_ref[...] = x_ref[...] * inv_s[:, None]


def _k1b(histr, hists, x):
    return pl.pallas_call(
        _norm_body,
        grid=(NB,),
        in_specs=[
            pl.BlockSpec((1, NW, BLK), lambda i: (i, 0, 0)),
            pl.BlockSpec((1, NW, BLK), lambda i: (i, 0, 0)),
            pl.BlockSpec((BLK, D_FEAT), lambda i: (i, 0)),
        ],
        out_specs=(
            pl.BlockSpec((BLK, D_FEAT), lambda i: (i, 0)),
            pl.BlockSpec((1, 1, BLK), lambda i: (i, 0, 0)),
        ),
        out_shape=(
            jax.ShapeDtypeStruct((N_NODES, D_FEAT), jnp.float32),
            jax.ShapeDtypeStruct((NB, 1, BLK), jnp.float32),
        ),
    )(histr, hists, x)


def _buf_set():
    return [
        pltpu.VMEM((CH,), jnp.int32),
        pltpu.VMEM((1, CH), jnp.int32),
        pltpu.VMEM((1, CH), jnp.int32),
        pltpu.VMEM((CH,), jnp.float32),
        pltpu.VMEM((CH, D_FEAT), jnp.float32),
        pltpu.SemaphoreType.DMA,
        pltpu.SemaphoreType.DMA,
        pltpu.SemaphoreType.DMA,
    ]


@functools.partial(
    pl.kernel,
    out_type=jax.ShapeDtypeStruct((NC, N_NODES, D_FEAT), jnp.float32),
    mesh=_mesh,
    scratch_types=[
        *_buf_set(), *_buf_set(), *_buf_set(),
        pltpu.VMEM_SHARED((N_NODES, D_FEAT), jnp.float32),
    ],
    compiler_params=_sc_params,
)
def _edge_kernel(xs_hbm, send_hbm, recv_hbm, w_hbm, p_hbm, *bufs):
    S = (bufs[0:8], bufs[8:16], bufs[16:24])
    acc = bufs[24]
    c = lax.axis_index("c")
    s = lax.axis_index("s")
    wid = c * NS + s

    zero = jnp.zeros((L,), jnp.float32)
    rows0 = S[0][4]

    @pl.loop(0, ZR)
    def _zb(i):
        for cc in range(D_FEAT // L):
            rows0[i, pl.ds(cc * L, L)] = zero

    rbase = s * RPS

    @pl.loop(0, RPS // ZR)
    def _za(i):
        pltpu.sync_copy(rows0.at[pl.ds(0, ZR)],
                        acc.at[pl.ds(rbase + i * ZR, ZR)])

    @pl.when(s == 0)
    def _za_tail():
        pltpu.sync_copy(rows0.at[pl.ds(0, TAIL)],
                        acc.at[pl.ds(NS * RPS, TAIL)])

    plsc.subcore_barrier()

    ebase = wid * EPW

    def issue_copies(n, bset):
        sidx, ridx, _, cf, _, _, isem, _ = bset
        base = ebase + n * CH
        pltpu.async_copy(send_hbm.at[pl.ds(base, CH)], sidx, isem)
        pltpu.async_copy(recv_hbm.at[pl.ds(base, CH)], ridx.at[0], isem)
        pltpu.async_copy(w_hbm.at[pl.ds(base, CH)], cf, isem)

    def wait_copies(bset):
        sidx, ridx, _, cf, _, _, isem, _ = bset
        pltpu.make_async_copy(send_hbm.at[pl.ds(0, CH)], sidx, isem).wait()
        pltpu.make_async_copy(recv_hbm.at[pl.ds(0, CH)], ridx.at[0],
                              isem).wait()
        pltpu.make_async_copy(w_hbm.at[pl.ds(0, CH)], cf, isem).wait()

    def issue_gather(bset):
        sidx, _, _, _, rows, gsem, _, _ = bset
        pltpu.async_copy(xs_hbm.at[sidx], rows, gsem)

    def wait_gather(bset):
        sidx, _, _, _, rows, gsem, _, _ = bset
        pltpu.make_async_copy(xs_hbm.at[sidx], rows, gsem).wait()

    def scale_and_scatter(bset):
        _, ridx, ridc, cf, rows, _, _, ssem = bset
        wait_gather(bset)

        @plsc.parallel_loop(0, CH, 1, unroll=8)
        def _scale(j):
            wspl = plsc.load_gather(cf, [jnp.zeros((L,), jnp.int32) + j])
            for cc in range(D_FEAT // L):
                sl = pl.ds(cc * L, L)
                rows[j, sl] = rows[j, sl] * wspl

        for k in range(CH // L):
            sl = pl.ds(k * L, L)
            ridc[0, sl] = ridx[0, sl]
        pltpu.async_copy(rows, acc.at[ridc.at[0]], ssem, add=True)

    def wait_scatter(bset):
        _, _, ridc, _, rows, _, _, ssem = bset
        pltpu.make_async_copy(rows, acc.at[ridc.at[0]], ssem).wait()

    def process(n, A, B, C):
        @pl.when(n + 1 < NCH)
        def _nxt():
            wait_copies(B)

        wait_scatter(B)

        @pl.when(n + 1 < NCH)
        def _nxt2():
            issue_gather(B)

        @pl.when(n + 2 < NCH)
        def _pre():
            issue_copies(n + 2, C)

        scale_and_scatter(A)

    issue_copies(0, S[0])
    wait_copies(S[0])
    issue_gather(S[0])
    issue_copies(1, S[1])
    wait_copies(S[1])
    issue_gather(S[1])
    issue_copies(2, S[2])
    scale_and_scatter(S[0])
    wait_copies(S[2])
    issue_gather(S[2])
    issue_copies(3, S[0])
    scale_and_scatter(S[1])

    @pl.loop(0, (NCH - 2) // 3)
    def _triple(j):
        n = j * 3 + 2
        process(n, S[2], S[0], S[1])
        process(n + 1, S[0], S[1], S[2])
        process(n + 2, S[1], S[2], S[0])

    wait_scatter(S[0])
    wait_scatter(S[1])

    plsc.subcore_barrier()
    pltpu.sync_copy(acc.at[pl.ds(rbase, RPS)], p_hbm.at[c, pl.ds(rbase, RPS)])

    @pl.when(s == 0)
    def _out_tail():
        pltpu.sync_copy(acc.at[pl.ds(NS * RPS, TAIL)],
                        p_hbm.at[c, pl.ds(NS * RPS, TAIL)])


def _mm_body(p_ref, iv_ref, w_ref, b_ref, o_ref):
    iv = iv_ref[0, 0]
    pooled = (p_ref[0] + p_ref[1]) * iv[:, None]
    o_ref[...] = (
        jnp.dot(pooled, w_ref[...], preferred_element_type=jnp.float32)
        + b_ref[...]
    )


def _k4(P, invr3, W, b2):
    return pl.pallas_call(
        _mm_body,
        grid=(NB,),
        in_specs=[
            pl.BlockSpec((NC, BLK, D_FEAT), lambda i: (0, i, 0)),
            pl.BlockSpec((1, 1, BLK), lambda i: (i, 0, 0)),
            pl.BlockSpec((D_FEAT, D_OUT), lambda i: (0, 0)),
            pl.BlockSpec((1, D_OUT), lambda i: (0, 0)),
        ],
        out_specs=pl.BlockSpec((BLK, D_OUT), lambda i: (i, 0)),
        out_shape=jax.ShapeDtypeStruct((N_NODES, D_OUT), jnp.float32),
    )(P, invr3, W, b2)


def kernel(x, edge_index, edge_weights, W, b):
    recv = edge_index[0]
    send = edge_index[1]
    histr, hists = _deg_kernel(recv, send, edge_weights)
    xs, invr3 = _k1b(histr, hists, x)
    P = _edge_kernel(xs, send, recv, edge_weights)
    out = _k4(P, invr3, W, b.reshape(1, D_OUT))
    return out


# speedup vs baseline: 1.0603x; 1.0603x over previous
import functools

import numpy as np

import jax
import jax.numpy as jnp
from jax import lax
from jax.experimental import pallas as pl
from jax.experimental.pallas import tpu as pltpu
from jax.experimental.pallas import tpu_sc as plsc

NC, NS, L = 2, 16, 16
NW = NC * NS

N_NODES = 10000
N_EDGES = 320000
D_FEAT = 128
D_OUT = 128

EPW = N_EDGES // NW
CH = 80
NCH = EPW // CH
RPS = 624
TAIL = N_NODES - NS * RPS
ZR = 78
NB = 10
BLK = N_NODES // NB

_mesh = plsc.VectorSubcoreMesh(core_axis_name="c", subcore_axis_name="s")
_sc_params = pltpu.CompilerParams(
    needs_layout_passes=False, use_tc_tiling_on_sc=False)


@functools.partial(
    pl.kernel,
    out_type=(
        jax.ShapeDtypeStruct((NB, NW, BLK), jnp.float32),
        jax.ShapeDtypeStruct((NB, NW, BLK), jnp.float32),
    ),
    mesh=_mesh,
    scratch_types=[
        pltpu.VMEM((EPW,), jnp.int32),
        pltpu.VMEM((EPW,), jnp.int32),
        pltpu.VMEM((EPW,), jnp.float32),
        pltpu.VMEM((N_NODES,), jnp.float32),
        pltpu.VMEM((N_NODES,), jnp.float32),
    ],
    compiler_params=pltpu.CompilerParams(
        needs_layout_passes=False, use_tc_tiling_on_sc=False),
)
def _deg_kernel(recv_hbm, send_hbm, w_hbm, histr_hbm, hists_hbm,
                ridx_v, sidx_v, w_v, hr, hs):
    c = lax.axis_index("c")
    s = lax.axis_index("s")
    wid = c * NS + s
    base = wid * EPW
    pltpu.sync_copy(recv_hbm.at[pl.ds(base, EPW)], ridx_v)
    pltpu.sync_copy(send_hbm.at[pl.ds(base, EPW)], sidx_v)
    pltpu.sync_copy(w_hbm.at[pl.ds(base, EPW)], w_v)

    zero = jnp.zeros((L,), jnp.float32)

    @pl.loop(0, N_NODES // L)
    def _zero(i):
        hr[pl.ds(i * L, L)] = zero
        hs[pl.ds(i * L, L)] = zero

    @pl.loop(0, EPW // L)
    def _acc(i):
        sl = pl.ds(i * L, L)
        wv = w_v[sl]
        plsc.addupdate_scatter(hr, [ridx_v[sl]], wv)
        plsc.addupdate_scatter(hs, [sidx_v[sl]], wv)

    for i in range(NB):
        pltpu.sync_copy(hr.at[pl.ds(i * BLK, BLK)], histr_hbm.at[i, wid])
        pltpu.sync_copy(hs.at[pl.ds(i * BLK, BLK)], hists_hbm.at[i, wid])


def _norm_body(hr_ref, hs_ref, x_ref, xs_ref, invr_ref):
    deg_r = jnp.sum(hr_ref[0], axis=0)
    safe_r = jnp.where(deg_r > 0, deg_r, 1.0)
    invr_ref[...] = jnp.where(deg_r > 0, lax.rsqrt(safe_r), 0.0)[None, None, :]
    deg_s = jnp.sum(hs_ref[0], axis=0)
    safe_s = jnp.where(deg_s > 0, deg_s, 1.0)
    inv_s = jnp.where(deg_s > 0, lax.rsqrt(safe_s), 0.0)
    xs_ref[...] = (x_ref[...] * inv_s[:, None]).astype(jnp.bfloat16)


def _k1b(histr, hists, x):
    return pl.pallas_call(
        _norm_body,
        grid=(NB,),
        in_specs=[
            pl.BlockSpec((1, NW, BLK), lambda i: (i, 0, 0)),
            pl.BlockSpec((1, NW, BLK), lambda i: (i, 0, 0)),
            pl.BlockSpec((BLK, D_FEAT), lambda i: (i, 0)),
        ],
        out_specs=(
            pl.BlockSpec((BLK, D_FEAT), lambda i: (i, 0)),
            pl.BlockSpec((1, 1, BLK), lambda i: (i, 0, 0)),
        ),
        out_shape=(
            jax.ShapeDtypeStruct((N_NODES, D_FEAT), jnp.bfloat16),
            jax.ShapeDtypeStruct((NB, 1, BLK), jnp.float32),
        ),
    )(histr, hists, x)


def _buf_set():
    return [
        pltpu.VMEM((CH,), jnp.int32),
        pltpu.VMEM((1, CH), jnp.int32),
        pltpu.VMEM((1, CH), jnp.int32),
        pltpu.VMEM((CH,), jnp.float32),
        pltpu.VMEM((CH, D_FEAT), jnp.bfloat16),
        pltpu.VMEM((CH, D_FEAT), jnp.float32),
        pltpu.SemaphoreType.DMA,
        pltpu.SemaphoreType.DMA,
        pltpu.SemaphoreType.DMA,
    ]


@functools.partial(
    pl.kernel,
    out_type=jax.ShapeDtypeStruct((NC, N_NODES, D_FEAT), jnp.float32),
    mesh=_mesh,
    scratch_types=[
        *_buf_set(), *_buf_set(), *_buf_set(),
        pltpu.VMEM_SHARED((N_NODES, D_FEAT), jnp.float32),
    ],
    compiler_params=_sc_params,
)
def _edge_kernel(xs_hbm, send_hbm, recv_hbm, w_hbm, p_hbm, *bufs):
    S = (bufs[0:9], bufs[9:18], bufs[18:27])
    acc = bufs[27]
    c = lax.axis_index("c")
    s = lax.axis_index("s")
    wid = c * NS + s

    zero = jnp.zeros((L,), jnp.float32)
    rows0 = S[0][5]

    @pl.loop(0, ZR)
    def _zb(i):
        for cc in range(D_FEAT // L):
            rows0[i, pl.ds(cc * L, L)] = zero

    rbase = s * RPS

    @pl.loop(0, RPS // ZR)
    def _za(i):
        pltpu.sync_copy(rows0.at[pl.ds(0, ZR)],
                        acc.at[pl.ds(rbase + i * ZR, ZR)])

    @pl.when(s == 0)
    def _za_tail():
        pltpu.sync_copy(rows0.at[pl.ds(0, TAIL)],
                        acc.at[pl.ds(NS * RPS, TAIL)])

    plsc.subcore_barrier()

    ebase = wid * EPW

    def issue_copies(n, bset):
        sidx, ridx, _, cf = bset[0:4]
        isem = bset[7]
        base = ebase + n * CH
        pltpu.async_copy(send_hbm.at[pl.ds(base, CH)], sidx, isem)
        pltpu.async_copy(recv_hbm.at[pl.ds(base, CH)], ridx.at[0], isem)
        pltpu.async_copy(w_hbm.at[pl.ds(base, CH)], cf, isem)

    def wait_copies(bset):
        sidx, ridx, _, cf = bset[0:4]
        isem = bset[7]
        pltpu.make_async_copy(send_hbm.at[pl.ds(0, CH)], sidx, isem).wait()
        pltpu.make_async_copy(recv_hbm.at[pl.ds(0, CH)], ridx.at[0],
                              isem).wait()
        pltpu.make_async_copy(w_hbm.at[pl.ds(0, CH)], cf, isem).wait()

    def issue_gather(bset):
        sidx, rows_bf, gsem = bset[0], bset[4], bset[6]
        pltpu.async_copy(xs_hbm.at[sidx], rows_bf, gsem)

    def wait_gather(bset):
        sidx, rows_bf, gsem = bset[0], bset[4], bset[6]
        pltpu.make_async_copy(xs_hbm.at[sidx], rows_bf, gsem).wait()

    def scale_and_scatter(bset):
        _, ridx, ridc, cf, rows_bf, rows, _, _, ssem = bset
        wait_gather(bset)

        @plsc.parallel_loop(0, CH, 1, unroll=8)
        def _scale(j):
            wspl = plsc.load_gather(cf, [jnp.zeros((L,), jnp.int32) + j])
            for cc in range(D_FEAT // (2 * L)):
                ab = rows_bf[j, pl.ds(cc * 2 * L, 2 * L)]
                a, b = plsc.unpack(ab, format=plsc.PackFormat.INTERLEAVED)
                rows[j, pl.ds(cc * 2 * L, L)] = a * wspl
                rows[j, pl.ds(cc * 2 * L + L, L)] = b * wspl

        for k in range(CH // L):
            sl = pl.ds(k * L, L)
            ridc[0, sl] = ridx[0, sl]
        pltpu.async_copy(rows, acc.at[ridc.at[0]], ssem, add=True)

    def wait_scatter(bset):
        ridc, rows, ssem = bset[2], bset[5], bset[8]
        pltpu.make_async_copy(rows, acc.at[ridc.at[0]], ssem).wait()

    def process(n, A, B, C):
        @pl.when(n + 1 < NCH)
        def _nxt():
            wait_copies(B)

        wait_scatter(B)

        @pl.when(n + 1 < NCH)
        def _nxt2():
            issue_gather(B)

        @pl.when(n + 2 < NCH)
        def _pre():
            issue_copies(n + 2, C)

        scale_and_scatter(A)

    issue_copies(0, S[0])
    wait_copies(S[0])
    issue_gather(S[0])
    issue_copies(1, S[1])
    wait_copies(S[1])
    issue_gather(S[1])
    issue_copies(2, S[2])
    scale_and_scatter(S[0])
    wait_copies(S[2])
    issue_gather(S[2])
    issue_copies(3, S[0])
    scale_and_scatter(S[1])

    @pl.loop(0, (NCH - 2) // 3)
    def _triple(j):
        n = j * 3 + 2
        process(n, S[2], S[0], S[1])
        process(n + 1, S[0], S[1], S[2])
        process(n + 2, S[1], S[2], S[0])

    wait_scatter(S[0])
    wait_scatter(S[1])

    plsc.subcore_barrier()
    pltpu.sync_copy(acc.at[pl.ds(rbase, RPS)], p_hbm.at[c, pl.ds(rbase, RPS)])

    @pl.when(s == 0)
    def _out_tail():
        pltpu.sync_copy(acc.at[pl.ds(NS * RPS, TAIL)],
                        p_hbm.at[c, pl.ds(NS * RPS, TAIL)])


def _mm_body(p_ref, iv_ref, w_ref, b_ref, o_ref):
    iv = iv_ref[0, 0]
    pooled = (p_ref[0] + p_ref[1]) * iv[:, None]
    o_ref[...] = (
        jnp.dot(pooled, w_ref[...], preferred_element_type=jnp.float32)
        + b_ref[...]
    )


def _k4(P, invr3, W, b2):
    return pl.pallas_call(
        _mm_body,
        grid=(NB,),
        in_specs=[
            pl.BlockSpec((NC, BLK, D_FEAT), lambda i: (0, i, 0)),
            pl.BlockSpec((1, 1, BLK), lambda i: (i, 0, 0)),
            pl.BlockSpec((D_FEAT, D_OUT), lambda i: (0, 0)),
            pl.BlockSpec((1, D_OUT), lambda i: (0, 0)),
        ],
        out_specs=pl.BlockSpec((BLK, D_OUT), lambda i: (i, 0)),
        out_shape=jax.ShapeDtypeStruct((N_NODES, D_OUT), jnp.float32),
    )(P, invr3, W, b2)


_PERM = np.asarray(
    [v for c in range(D_FEAT // 32)
     for v in ([c * 32 + 2 * i for i in range(16)]
               + [c * 32 + 2 * i + 1 for i in range(16)])],
    dtype=np.int32)


def kernel(x, edge_index, edge_weights, W, b):
    recv = edge_index[0]
    send = edge_index[1]
    histr, hists = _deg_kernel(recv, send, edge_weights)
    xs, invr3 = _k1b(histr, hists, x)
    P = _edge_kernel(xs, send, recv, edge_weights)
    out = _k4(P, invr3, W[_PERM, :], b.reshape(1, D_OUT))
    return out


# speedup vs baseline: 1.0618x; 1.0014x over previous
import functools

import numpy as np

import jax
import jax.numpy as jnp
from jax import lax
from jax.experimental import pallas as pl
from jax.experimental.pallas import tpu as pltpu
from jax.experimental.pallas import tpu_sc as plsc

NC, NS, L = 2, 16, 16
NW = NC * NS

N_NODES = 10000
N_EDGES = 320000
D_FEAT = 128
D_OUT = 128

EPW = N_EDGES // NW
CH = 80
NCH = EPW // CH
RPS = 624
TAIL = N_NODES - NS * RPS
ZR = 78
NB = 10
BLK = N_NODES // NB

_mesh = plsc.VectorSubcoreMesh(core_axis_name="c", subcore_axis_name="s")
_sc_params = pltpu.CompilerParams(
    needs_layout_passes=False, use_tc_tiling_on_sc=False)


@functools.partial(
    pl.kernel,
    out_type=(
        jax.ShapeDtypeStruct((NB, NW, BLK), jnp.float32),
        jax.ShapeDtypeStruct((NB, NW, BLK), jnp.float32),
    ),
    mesh=_mesh,
    scratch_types=[
        pltpu.VMEM((EPW,), jnp.int32),
        pltpu.VMEM((EPW,), jnp.int32),
        pltpu.VMEM((EPW,), jnp.float32),
        pltpu.VMEM((N_NODES,), jnp.float32),
        pltpu.VMEM((N_NODES,), jnp.float32),
    ],
    compiler_params=pltpu.CompilerParams(
        needs_layout_passes=False, use_tc_tiling_on_sc=False),
)
def _deg_kernel(recv_hbm, send_hbm, w_hbm, histr_hbm, hists_hbm,
                ridx_v, sidx_v, w_v, hr, hs):
    c = lax.axis_index("c")
    s = lax.axis_index("s")
    wid = c * NS + s
    base = wid * EPW
    pltpu.sync_copy(recv_hbm.at[pl.ds(base, EPW)], ridx_v)
    pltpu.sync_copy(send_hbm.at[pl.ds(base, EPW)], sidx_v)
    pltpu.sync_copy(w_hbm.at[pl.ds(base, EPW)], w_v)

    zero = jnp.zeros((L,), jnp.float32)

    @pl.loop(0, N_NODES // L)
    def _zero(i):
        hr[pl.ds(i * L, L)] = zero
        hs[pl.ds(i * L, L)] = zero

    @pl.loop(0, EPW // L)
    def _acc(i):
        sl = pl.ds(i * L, L)
        wv = w_v[sl]
        plsc.addupdate_scatter(hr, [ridx_v[sl]], wv)
        plsc.addupdate_scatter(hs, [sidx_v[sl]], wv)

    for i in range(NB):
        pltpu.sync_copy(hr.at[pl.ds(i * BLK, BLK)], histr_hbm.at[i, wid])
        pltpu.sync_copy(hs.at[pl.ds(i * BLK, BLK)], hists_hbm.at[i, wid])


def _norm_body(hr_ref, hs_ref, x_ref, xs_ref, invr_ref):
    deg_r = jnp.sum(hr_ref[0], axis=0)
    safe_r = jnp.where(deg_r > 0, deg_r, 1.0)
    invr_ref[...] = jnp.where(deg_r > 0, lax.rsqrt(safe_r), 0.0)[None, None, :]
    deg_s = jnp.sum(hs_ref[0], axis=0)
    safe_s = jnp.where(deg_s > 0, deg_s, 1.0)
    inv_s = jnp.where(deg_s > 0, lax.rsqrt(safe_s), 0.0)
    xs_ref[...] = (x_ref[...] * inv_s[:, None]).astype(jnp.bfloat16)


def _k1b(histr, hists, x):
    return pl.pallas_call(
        _norm_body,
        grid=(NB,),
        in_specs=[
            pl.BlockSpec((1, NW, BLK), lambda i: (i, 0, 0)),
            pl.BlockSpec((1, NW, BLK), lambda i: (i, 0, 0)),
            pl.BlockSpec((BLK, D_FEAT), lambda i: (i, 0)),
        ],
        out_specs=(
            pl.BlockSpec((BLK, D_FEAT), lambda i: (i, 0)),
            pl.BlockSpec((1, 1, BLK), lambda i: (i, 0, 0)),
        ),
        out_shape=(
            jax.ShapeDtypeStruct((N_NODES, D_FEAT), jnp.bfloat16),
            jax.ShapeDtypeStruct((NB, 1, BLK), jnp.float32),
        ),
    )(histr, hists, x)


def _buf_set():
    return [
        pltpu.VMEM((CH,), jnp.int32),
        pltpu.VMEM((1, CH), jnp.int32),
        pltpu.VMEM((1, CH), jnp.int32),
        pltpu.VMEM((CH,), jnp.float32),
        pltpu.VMEM((CH, D_FEAT), jnp.bfloat16),
        pltpu.VMEM((CH, D_FEAT), jnp.float32),
        pltpu.SemaphoreType.DMA,
        pltpu.SemaphoreType.DMA,
        pltpu.SemaphoreType.DMA,
    ]


@functools.partial(
    pl.kernel,
    out_type=jax.ShapeDtypeStruct((NC, N_NODES, D_FEAT), jnp.float32),
    mesh=_mesh,
    scratch_types=[
        *_buf_set(), *_buf_set(), *_buf_set(),
        pltpu.VMEM_SHARED((N_NODES, D_FEAT), jnp.float32),
    ],
    compiler_params=_sc_params,
)
def _edge_kernel(xs_hbm, send_hbm, recv_hbm, w_hbm, p_hbm, *bufs):
    S = (bufs[0:9], bufs[9:18], bufs[18:27])
    acc = bufs[27]
    c = lax.axis_index("c")
    s = lax.axis_index("s")
    wid = c * NS + s

    zero = jnp.zeros((L,), jnp.float32)
    rows0 = S[0][5]

    @pl.loop(0, ZR)
    def _zb(i):
        for cc in range(D_FEAT // L):
            rows0[i, pl.ds(cc * L, L)] = zero

    rbase = s * RPS

    @pl.loop(0, RPS // ZR)
    def _za(i):
        pltpu.sync_copy(rows0.at[pl.ds(0, ZR)],
                        acc.at[pl.ds(rbase + i * ZR, ZR)])

    @pl.when(s == 0)
    def _za_tail():
        pltpu.sync_copy(rows0.at[pl.ds(0, TAIL)],
                        acc.at[pl.ds(NS * RPS, TAIL)])

    plsc.subcore_barrier()

    ebase = wid * EPW

    def issue_copies(n, bset):
        sidx, ridx, _, cf = bset[0:4]
        isem = bset[7]
        base = ebase + n * CH
        pltpu.async_copy(send_hbm.at[pl.ds(base, CH)], sidx, isem)
        pltpu.async_copy(recv_hbm.at[pl.ds(base, CH)], ridx.at[0], isem)
        pltpu.async_copy(w_hbm.at[pl.ds(base, CH)], cf, isem)

    def wait_copies(bset):
        sidx, ridx, _, cf = bset[0:4]
        isem = bset[7]
        pltpu.make_async_copy(send_hbm.at[pl.ds(0, CH)], sidx, isem).wait()
        pltpu.make_async_copy(recv_hbm.at[pl.ds(0, CH)], ridx.at[0],
                              isem).wait()
        pltpu.make_async_copy(w_hbm.at[pl.ds(0, CH)], cf, isem).wait()

    def issue_gather(bset):
        sidx, rows_bf, gsem = bset[0], bset[4], bset[6]
        pltpu.async_copy(xs_hbm.at[sidx], rows_bf, gsem)

    def wait_gather(bset):
        sidx, rows_bf, gsem = bset[0], bset[4], bset[6]
        pltpu.make_async_copy(xs_hbm.at[sidx], rows_bf, gsem).wait()

    def scale_and_scatter(bset):
        _, ridx, ridc, cf, rows_bf, rows, _, _, ssem = bset
        wait_gather(bset)

        @plsc.parallel_loop(0, CH, 1, unroll=16)
        def _scale(j):
            wspl = plsc.load_gather(cf, [jnp.zeros((L,), jnp.int32) + j])
            for cc in range(D_FEAT // (2 * L)):
                ab = rows_bf[j, pl.ds(cc * 2 * L, 2 * L)]
                a, b = plsc.unpack(ab, format=plsc.PackFormat.INTERLEAVED)
                rows[j, pl.ds(cc * 2 * L, L)] = a * wspl
                rows[j, pl.ds(cc * 2 * L + L, L)] = b * wspl

        for k in range(CH // L):
            sl = pl.ds(k * L, L)
            ridc[0, sl] = ridx[0, sl]
        pltpu.async_copy(rows, acc.at[ridc.at[0]], ssem, add=True)

    def wait_scatter(bset):
        ridc, rows, ssem = bset[2], bset[5], bset[8]
        pltpu.make_async_copy(rows, acc.at[ridc.at[0]], ssem).wait()

    def process(n, A, B, C):
        @pl.when(n + 1 < NCH)
        def _nxt():
            wait_copies(B)

        wait_scatter(B)

        @pl.when(n + 1 < NCH)
        def _nxt2():
            issue_gather(B)

        @pl.when(n + 2 < NCH)
        def _pre():
            issue_copies(n + 2, C)

        scale_and_scatter(A)

    issue_copies(0, S[0])
    wait_copies(S[0])
    issue_gather(S[0])
    issue_copies(1, S[1])
    wait_copies(S[1])
    issue_gather(S[1])
    issue_copies(2, S[2])
    scale_and_scatter(S[0])
    wait_copies(S[2])
    issue_gather(S[2])
    issue_copies(3, S[0])
    scale_and_scatter(S[1])

    @pl.loop(0, (NCH - 2) // 3)
    def _triple(j):
        n = j * 3 + 2
        process(n, S[2], S[0], S[1])
        process(n + 1, S[0], S[1], S[2])
        process(n + 2, S[1], S[2], S[0])

    wait_scatter(S[0])
    wait_scatter(S[1])

    plsc.subcore_barrier()
    pltpu.sync_copy(acc.at[pl.ds(rbase, RPS)], p_hbm.at[c, pl.ds(rbase, RPS)])

    @pl.when(s == 0)
    def _out_tail():
        pltpu.sync_copy(acc.at[pl.ds(NS * RPS, TAIL)],
                        p_hbm.at[c, pl.ds(NS * RPS, TAIL)])


def _mm_body(p_ref, iv_ref, w_ref, b_ref, o_ref):
    iv = iv_ref[0, 0]
    pooled = (p_ref[0] + p_ref[1]) * iv[:, None]
    o_ref[...] = (
        jnp.dot(pooled, w_ref[...], preferred_element_type=jnp.float32)
        + b_ref[...]
    )


def _k4(P, invr3, W, b2):
    return pl.pallas_call(
        _mm_body,
        grid=(NB,),
        in_specs=[
            pl.BlockSpec((NC, BLK, D_FEAT), lambda i: (0, i, 0)),
            pl.BlockSpec((1, 1, BLK), lambda i: (i, 0, 0)),
            pl.BlockSpec((D_FEAT, D_OUT), lambda i: (0, 0)),
            pl.BlockSpec((1, D_OUT), lambda i: (0, 0)),
        ],
        out_specs=pl.BlockSpec((BLK, D_OUT), lambda i: (i, 0)),
        out_shape=jax.ShapeDtypeStruct((N_NODES, D_OUT), jnp.float32),
    )(P, invr3, W, b2)


_PERM = np.asarray(
    [v for c in range(D_FEAT // 32)
     for v in ([c * 32 + 2 * i for i in range(16)]
               + [c * 32 + 2 * i + 1 for i in range(16)])],
    dtype=np.int32)


def kernel(x, edge_index, edge_weights, W, b):
    recv = edge_index[0]
    send = edge_index[1]
    histr, hists = _deg_kernel(recv, send, edge_weights)
    xs, invr3 = _k1b(histr, hists, x)
    P = _edge_kernel(xs, send, recv, edge_weights)
    out = _k4(P, invr3, W[_PERM, :], b.reshape(1, D_OUT))
    return out
